# baseline (speedup 1.0000x reference)
import functools

import jax
import jax.numpy as jnp
from jax import lax
from jax.experimental import pallas as pl
from jax.experimental.pallas import tpu as pltpu
from jax.experimental.pallas import tpu_sc as plsc

_U, _V, _E = 10000, 10000, 160000
_F, _G, _H, _D = 256, 256, 16, 256
_HID = 512

_NC, _NS, _L = 2, 16, 16
_NW = _NC * _NS

_EPW = _E // _NW
_KG = 40
_NBG = _EPW // _KG

_CH = _D // _NC
_EPS = _E // _NS
_K2 = 80
_NB2 = _EPS // _K2
_ACC = 10240
_RPS = _ACC // _NS
_TAIL = _U - (_NS - 1) * _RPS


def _sc_mesh():
    return plsc.VectorSubcoreMesh(
        core_axis_name="c", subcore_axis_name="s", num_cores=_NC, num_subcores=_NS
    )


def _pre_body(u_ref, v_ref, wu_ref, wv_ref, b1_ref, a_ref, b_ref):
    a_ref[...] = (
        jnp.dot(u_ref[...], wu_ref[...], preferred_element_type=jnp.float32)
        + b1_ref[...]
    ).astype(jnp.bfloat16)
    b_ref[...] = jnp.dot(
        v_ref[...], wv_ref[...], preferred_element_type=jnp.float32
    ).astype(jnp.bfloat16)


def _precompute(u, v, wu, wv, b1):
    rb = 1000
    return pl.pallas_call(
        _pre_body,
        grid=(_U // rb,),
        in_specs=[
            pl.BlockSpec((rb, _F), lambda i: (i, 0)),
            pl.BlockSpec((rb, _G), lambda i: (i, 0)),
            pl.BlockSpec((_F, _HID), lambda i: (0, 0)),
            pl.BlockSpec((_G, _HID), lambda i: (0, 0)),
            pl.BlockSpec((1, _HID), lambda i: (0, 0)),
        ],
        out_specs=[
            pl.BlockSpec((rb, _HID), lambda i: (i, 0)),
            pl.BlockSpec((rb, _HID), lambda i: (i, 0)),
        ],
        out_shape=[
            jax.ShapeDtypeStruct((_U, _HID), jnp.bfloat16),
            jax.ShapeDtypeStruct((_V, _HID), jnp.bfloat16),
        ],
    )(u, v, wu, wv, b1)


_HID32 = _HID // 2


def _gather_body(
    a_hbm, b_hbm, ui_hbm, vi_hbm, sa_hbm, sb_hbm, idxu, idxv, ra, rb, sga, sgb
):
    wid = lax.axis_index("s") * _NC + lax.axis_index("c")
    base = wid * _EPW

    def start(j, k):
        off = pl.multiple_of(base + j * _KG, _KG)
        pltpu.sync_copy(ui_hbm.at[pl.ds(off, _KG)], idxu[k])
        pltpu.sync_copy(vi_hbm.at[pl.ds(off, _KG)], idxv[k])
        pltpu.async_copy(a_hbm.at[idxu[k]], ra[k], sga[k])
        pltpu.async_copy(b_hbm.at[idxv[k]], rb[k], sgb[k])

    def finish(j, k):
        off = pl.multiple_of(base + j * _KG, _KG)
        pltpu.make_async_copy(a_hbm.at[idxu[k]], ra[k], sga[k]).wait()
        pltpu.make_async_copy(b_hbm.at[idxv[k]], rb[k], sgb[k]).wait()
        pltpu.sync_copy(ra[k], sa_hbm.at[pl.ds(off, _KG)])
        pltpu.sync_copy(rb[k], sb_hbm.at[pl.ds(off, _KG)])

    start(0, 0)

    def pair(p, carry):
        j0 = p * 2
        start(j0 + 1, 1)
        finish(j0, 0)
        start(j0 + 2, 0)
        finish(j0 + 1, 1)
        return carry

    lax.fori_loop(0, (_NBG - 1) // 2, pair, 0)
    finish(_NBG - 1, 0)


def _gather_pair(a, b, ui, vi):
    def body(a_hbm, b_hbm, ui_hbm, vi_hbm, sa_hbm, sb_hbm, iu0, iu1, iv0, iv1,
             ra0, ra1, rb0, rb1, sa0, sa1, sb0, sb1):
        _gather_body(
            a_hbm, b_hbm, ui_hbm, vi_hbm, sa_hbm, sb_hbm,
            [iu0, iu1], [iv0, iv1], [ra0, ra1], [rb0, rb1],
            [sa0, sa1], [sb0, sb1],
        )

    fn = pl.kernel(
        body,
        out_type=[
            jax.ShapeDtypeStruct((_E, _HID32), jnp.int32),
            jax.ShapeDtypeStruct((_E, _HID32), jnp.int32),
        ],
        mesh=_sc_mesh(),
        scratch_types=[
            pltpu.VMEM((_KG,), jnp.int32),
            pltpu.VMEM((_KG,), jnp.int32),
            pltpu.VMEM((_KG,), jnp.int32),
            pltpu.VMEM((_KG,), jnp.int32),
            pltpu.VMEM((_KG, _HID32), jnp.int32),
            pltpu.VMEM((_KG, _HID32), jnp.int32),
            pltpu.VMEM((_KG, _HID32), jnp.int32),
            pltpu.VMEM((_KG, _HID32), jnp.int32),
            pltpu.SemaphoreType.DMA,
            pltpu.SemaphoreType.DMA,
            pltpu.SemaphoreType.DMA,
            pltpu.SemaphoreType.DMA,
        ],
    )
    return fn(a, b, ui, vi)


def _mlp_body(sa_ref, sb_ref, ev_ref, we_ref, w2_ref, b2_ref, g_ref):
    h = (
        sa_ref[...].astype(jnp.float32)
        + sb_ref[...].astype(jnp.float32)
        + jnp.dot(ev_ref[...], we_ref[...], preferred_element_type=jnp.float32)
    )
    h = jnp.maximum(h, 0.0).astype(jnp.bfloat16)
    g = (
        jnp.dot(h, w2_ref[...], preferred_element_type=jnp.float32)
        + b2_ref[...]
    )
    g_ref[...] = jnp.maximum(g, 0.0)


def _edge_mlp(sa, sb, ev, we, w2, b2):
    be = 1280
    return pl.pallas_call(
        _mlp_body,
        grid=(_E // be,),
        in_specs=[
            pl.BlockSpec((be, _HID), lambda i: (i, 0)),
            pl.BlockSpec((be, _HID), lambda i: (i, 0)),
            pl.BlockSpec((be, _H), lambda i: (i, 0)),
            pl.BlockSpec((_H, _HID), lambda i: (0, 0)),
            pl.BlockSpec((_HID, _D), lambda i: (0, 0)),
            pl.BlockSpec((1, _D), lambda i: (0, 0)),
        ],
        out_specs=pl.BlockSpec((be, _D), lambda i: (i, 0)),
        out_shape=jax.ShapeDtypeStruct((_E, _D), jnp.float32),
    )(sa, sb, ev, we, w2, b2)


def _scatter_body(g_hbm, ui_hbm, out_hbm, idx, rows, acc):
    cid = lax.axis_index("c")
    sid = lax.axis_index("s")
    col = pl.multiple_of(cid * _CH, _CH)
    rbase = pl.multiple_of(sid * _RPS, _RPS)
    zero = jnp.zeros((_L,), jnp.float32)

    def zrow(r, carry):
        for c in range(_CH // _L):
            rows[r, pl.ds(c * _L, _L)] = zero
        return carry

    lax.fori_loop(0, _K2, zrow, 0)
    for k in range(_RPS // _K2):
        pltpu.sync_copy(rows, acc.at[pl.ds(rbase + k * _K2, _K2)])
    plsc.subcore_barrier()

    def blk(j, carry):
        off = pl.multiple_of(sid * _EPS + j * _K2, _K2)
        pltpu.sync_copy(ui_hbm.at[pl.ds(off, _K2)], idx)
        pltpu.sync_copy(g_hbm.at[pl.ds(off, _K2), pl.ds(col, _CH)], rows)
        pltpu.sync_copy(rows, acc.at[idx], add=True)
        return carry

    lax.fori_loop(0, _NB2, blk, 0)
    plsc.subcore_barrier()

    @pl.when(sid < _NS - 1)
    def _copy_full():
        pltpu.sync_copy(
            acc.at[pl.ds(rbase, _RPS)], out_hbm.at[pl.ds(rbase, _RPS), pl.ds(col, _CH)]
        )

    @pl.when(sid == _NS - 1)
    def _copy_tail():
        tb = (_NS - 1) * _RPS
        pltpu.sync_copy(
            acc.at[pl.ds(tb, _TAIL)], out_hbm.at[pl.ds(tb, _TAIL), pl.ds(col, _CH)]
        )


def _segment_sum(g, ui):
    fn = pl.kernel(
        _scatter_body,
        out_type=jax.ShapeDtypeStruct((_U, _D), jnp.float32),
        mesh=_sc_mesh(),
        scratch_types=[
            pltpu.VMEM((_K2,), jnp.int32),
            pltpu.VMEM((_K2, _CH), jnp.float32),
            pltpu.VMEM_SHARED((_ACC, _CH), jnp.float32),
        ],
    )
    return fn(g, ui)


def _fin_body(u_ref, agg_ref, wu_ref, wa_ref, b_ref, o_ref):
    o = (
        jnp.dot(
            u_ref[...].astype(jnp.bfloat16),
            wu_ref[...],
            preferred_element_type=jnp.float32,
        )
        + jnp.dot(
            agg_ref[...].astype(jnp.bfloat16),
            wa_ref[...],
            preferred_element_type=jnp.float32,
        )
        + b_ref[...]
    )
    o_ref[...] = jnp.maximum(o, 0.0)


def _final(u, agg, wu, wa, b):
    rb = 1000
    return pl.pallas_call(
        _fin_body,
        grid=(_U // rb,),
        in_specs=[
            pl.BlockSpec((rb, _F), lambda i: (i, 0)),
            pl.BlockSpec((rb, _D), lambda i: (i, 0)),
            pl.BlockSpec((_F, _D), lambda i: (0, 0)),
            pl.BlockSpec((_D, _D), lambda i: (0, 0)),
            pl.BlockSpec((1, _D), lambda i: (0, 0)),
        ],
        out_specs=pl.BlockSpec((rb, _D), lambda i: (i, 0)),
        out_shape=jax.ShapeDtypeStruct((_U, _D), jnp.float32),
    )(u, agg, wu, wa, b)


def kernel(u, v, e_indices, e_values, Wg1, bg1, Wg2, bg2, Wf1, bf1):
    vi = e_indices[0]
    ui = e_indices[1]
    wu = Wg1[:_F]
    wv = Wg1[_F : _F + _G]
    we = Wg1[_F + _G :]
    a, b = _precompute(u, v, wu, wv, bg1.reshape(1, _HID))
    a32 = lax.bitcast_convert_type(a.reshape(_U, _HID32, 2), jnp.int32)
    b32 = lax.bitcast_convert_type(b.reshape(_V, _HID32, 2), jnp.int32)
    sa32, sb32 = _gather_pair(a32, b32, ui, vi)
    sa = lax.bitcast_convert_type(sa32, jnp.bfloat16).reshape(_E, _HID)
    sb = lax.bitcast_convert_type(sb32, jnp.bfloat16).reshape(_E, _HID)
    g = _edge_mlp(
        sa, sb, e_values, we, Wg2.astype(jnp.bfloat16), bg2.reshape(1, _D)
    )
    agg = _segment_sum(g, ui)
    return _final(
        u,
        agg,
        Wf1[:_F].astype(jnp.bfloat16),
        Wf1[_F:].astype(jnp.bfloat16),
        bf1.reshape(1, _D),
    )


# speedup vs baseline: 4.5932x; 4.5932x over previous
import functools

import jax
import jax.numpy as jnp
from jax import lax
from jax.experimental import pallas as pl
from jax.experimental.pallas import tpu as pltpu
from jax.experimental.pallas import tpu_sc as plsc

_U, _V, _E = 10000, 10000, 160000
_F, _G, _H, _D = 256, 256, 16, 256
_HID = 512

_NC, _NS, _L = 2, 16, 16
_NW = _NC * _NS

_EPW = _E // _NW
_KG = 40
_NBG = _EPW // _KG

_CH = _D // _NC
_EPS = _E // _NS
_K2 = 80
_NB2 = _EPS // _K2
_ACC = 10240
_RPS = _ACC // _NS
_TAIL = _U - (_NS - 1) * _RPS


def _sc_mesh():
    return plsc.VectorSubcoreMesh(
        core_axis_name="c", subcore_axis_name="s", num_cores=_NC, num_subcores=_NS
    )


_HID32 = _HID // 2


def _pack_bf16_pair(x):
    u = pltpu.bitcast(x, jnp.int32)
    r = u + jnp.int32(0x7FFF) + ((u >> 16) & 1)
    lo = r[:, :_HID32]
    hi = r[:, _HID32:]
    return ((lo >> 16) & jnp.int32(0xFFFF)) | (hi & jnp.int32(-65536))


def _unpack_lo(x):
    return pltpu.bitcast(x << 16, jnp.float32)


def _unpack_hi(x):
    return pltpu.bitcast(x & jnp.int32(-65536), jnp.float32)


def _pre_body(u_ref, v_ref, wu_ref, wv_ref, b1_ref, a_ref, b_ref):
    a_ref[...] = _pack_bf16_pair(
        jnp.dot(u_ref[...], wu_ref[...], preferred_element_type=jnp.float32)
        + b1_ref[...]
    )
    b_ref[...] = _pack_bf16_pair(
        jnp.dot(v_ref[...], wv_ref[...], preferred_element_type=jnp.float32)
    )


def _precompute(u, v, wu, wv, b1):
    rb = 1000
    return pl.pallas_call(
        _pre_body,
        grid=(_U // rb,),
        in_specs=[
            pl.BlockSpec((rb, _F), lambda i: (i, 0)),
            pl.BlockSpec((rb, _G), lambda i: (i, 0)),
            pl.BlockSpec((_F, _HID), lambda i: (0, 0)),
            pl.BlockSpec((_G, _HID), lambda i: (0, 0)),
            pl.BlockSpec((1, _HID), lambda i: (0, 0)),
        ],
        out_specs=[
            pl.BlockSpec((rb, _HID32), lambda i: (i, 0)),
            pl.BlockSpec((rb, _HID32), lambda i: (i, 0)),
        ],
        out_shape=[
            jax.ShapeDtypeStruct((_U, _HID32), jnp.int32),
            jax.ShapeDtypeStruct((_V, _HID32), jnp.int32),
        ],
    )(u, v, wu, wv, b1)


def _gather_body(
    a_hbm, b_hbm, ui_hbm, vi_hbm, sa_hbm, sb_hbm, idxu, idxv, ra, rb, sga, sgb
):
    wid = lax.axis_index("s") * _NC + lax.axis_index("c")
    base = wid * _EPW

    def start(j, k):
        off = pl.multiple_of(base + j * _KG, _KG)
        pltpu.sync_copy(ui_hbm.at[pl.ds(off, _KG)], idxu[k])
        pltpu.sync_copy(vi_hbm.at[pl.ds(off, _KG)], idxv[k])
        pltpu.async_copy(a_hbm.at[idxu[k]], ra[k], sga[k])
        pltpu.async_copy(b_hbm.at[idxv[k]], rb[k], sgb[k])

    def finish(j, k):
        off = pl.multiple_of(base + j * _KG, _KG)
        pltpu.make_async_copy(a_hbm.at[idxu[k]], ra[k], sga[k]).wait()
        pltpu.make_async_copy(b_hbm.at[idxv[k]], rb[k], sgb[k]).wait()
        pltpu.sync_copy(ra[k], sa_hbm.at[pl.ds(off, _KG)])
        pltpu.sync_copy(rb[k], sb_hbm.at[pl.ds(off, _KG)])

    start(0, 0)

    def pair(p, carry):
        j0 = p * 2
        start(j0 + 1, 1)
        finish(j0, 0)
        start(j0 + 2, 0)
        finish(j0 + 1, 1)
        return carry

    lax.fori_loop(0, (_NBG - 1) // 2, pair, 0)
    finish(_NBG - 1, 0)


def _gather_pair(a, b, ui, vi):
    def body(a_hbm, b_hbm, ui_hbm, vi_hbm, sa_hbm, sb_hbm, iu0, iu1, iv0, iv1,
             ra0, ra1, rb0, rb1, sa0, sa1, sb0, sb1):
        _gather_body(
            a_hbm, b_hbm, ui_hbm, vi_hbm, sa_hbm, sb_hbm,
            [iu0, iu1], [iv0, iv1], [ra0, ra1], [rb0, rb1],
            [sa0, sa1], [sb0, sb1],
        )

    fn = pl.kernel(
        body,
        out_type=[
            jax.ShapeDtypeStruct((_E, _HID32), jnp.int32),
            jax.ShapeDtypeStruct((_E, _HID32), jnp.int32),
        ],
        mesh=_sc_mesh(),
        scratch_types=[
            pltpu.VMEM((_KG,), jnp.int32),
            pltpu.VMEM((_KG,), jnp.int32),
            pltpu.VMEM((_KG,), jnp.int32),
            pltpu.VMEM((_KG,), jnp.int32),
            pltpu.VMEM((_KG, _HID32), jnp.int32),
            pltpu.VMEM((_KG, _HID32), jnp.int32),
            pltpu.VMEM((_KG, _HID32), jnp.int32),
            pltpu.VMEM((_KG, _HID32), jnp.int32),
            pltpu.SemaphoreType.DMA,
            pltpu.SemaphoreType.DMA,
            pltpu.SemaphoreType.DMA,
            pltpu.SemaphoreType.DMA,
        ],
    )
    return fn(a, b, ui, vi)


def _mlp_body(sa_ref, sb_ref, ev_ref, we_ref, w2lo_ref, w2hi_ref, b2_ref, g_ref):
    ew = jnp.dot(ev_ref[...], we_ref[...], preferred_element_type=jnp.float32)
    xa = sa_ref[...]
    xb = sb_ref[...]
    hlo = _unpack_lo(xa) + _unpack_lo(xb) + ew[:, :_HID32]
    hhi = _unpack_hi(xa) + _unpack_hi(xb) + ew[:, _HID32:]
    hlo = jnp.maximum(hlo, 0.0).astype(jnp.bfloat16)
    hhi = jnp.maximum(hhi, 0.0).astype(jnp.bfloat16)
    g = (
        jnp.dot(hlo, w2lo_ref[...], preferred_element_type=jnp.float32)
        + jnp.dot(hhi, w2hi_ref[...], preferred_element_type=jnp.float32)
        + b2_ref[...]
    )
    g_ref[...] = jnp.maximum(g, 0.0)


def _edge_mlp(sa, sb, ev, we, w2lo, w2hi, b2):
    be = 1280
    return pl.pallas_call(
        _mlp_body,
        grid=(_E // be,),
        in_specs=[
            pl.BlockSpec((be, _HID32), lambda i: (i, 0)),
            pl.BlockSpec((be, _HID32), lambda i: (i, 0)),
            pl.BlockSpec((be, _H), lambda i: (i, 0)),
            pl.BlockSpec((_H, _HID), lambda i: (0, 0)),
            pl.BlockSpec((_HID32, _D), lambda i: (0, 0)),
            pl.BlockSpec((_HID32, _D), lambda i: (0, 0)),
            pl.BlockSpec((1, _D), lambda i: (0, 0)),
        ],
        out_specs=pl.BlockSpec((be, _D), lambda i: (i, 0)),
        out_shape=jax.ShapeDtypeStruct((_E, _D), jnp.float32),
    )(sa, sb, ev, we, w2lo, w2hi, b2)


def _scatter_body(g_hbm, ui_hbm, out_hbm, idx, rows, acc):
    cid = lax.axis_index("c")
    sid = lax.axis_index("s")
    col = pl.multiple_of(cid * _CH, _CH)
    rbase = pl.multiple_of(sid * _RPS, _RPS)
    zero = jnp.zeros((_L,), jnp.float32)

    def zrow(r, carry):
        for c in range(_CH // _L):
            rows[r, pl.ds(c * _L, _L)] = zero
        return carry

    lax.fori_loop(0, _K2, zrow, 0)
    for k in range(_RPS // _K2):
        pltpu.sync_copy(rows, acc.at[pl.ds(rbase + k * _K2, _K2)])
    plsc.subcore_barrier()

    def blk(j, carry):
        off = pl.multiple_of(sid * _EPS + j * _K2, _K2)
        pltpu.sync_copy(ui_hbm.at[pl.ds(off, _K2)], idx)
        pltpu.sync_copy(g_hbm.at[pl.ds(off, _K2), pl.ds(col, _CH)], rows)
        pltpu.sync_copy(rows, acc.at[idx], add=True)
        return carry

    lax.fori_loop(0, _NB2, blk, 0)
    plsc.subcore_barrier()

    @pl.when(sid < _NS - 1)
    def _copy_full():
        pltpu.sync_copy(
            acc.at[pl.ds(rbase, _RPS)], out_hbm.at[pl.ds(rbase, _RPS), pl.ds(col, _CH)]
        )

    @pl.when(sid == _NS - 1)
    def _copy_tail():
        tb = (_NS - 1) * _RPS
        pltpu.sync_copy(
            acc.at[pl.ds(tb, _TAIL)], out_hbm.at[pl.ds(tb, _TAIL), pl.ds(col, _CH)]
        )


def _segment_sum(g, ui):
    fn = pl.kernel(
        _scatter_body,
        out_type=jax.ShapeDtypeStruct((_U, _D), jnp.float32),
        mesh=_sc_mesh(),
        scratch_types=[
            pltpu.VMEM((_K2,), jnp.int32),
            pltpu.VMEM((_K2, _CH), jnp.float32),
            pltpu.VMEM_SHARED((_ACC, _CH), jnp.float32),
        ],
    )
    return fn(g, ui)


def _fin_body(u_ref, agg_ref, wu_ref, wa_ref, b_ref, o_ref):
    o = (
        jnp.dot(
            u_ref[...].astype(jnp.bfloat16),
            wu_ref[...],
            preferred_element_type=jnp.float32,
        )
        + jnp.dot(
            agg_ref[...].astype(jnp.bfloat16),
            wa_ref[...],
            preferred_element_type=jnp.float32,
        )
        + b_ref[...]
    )
    o_ref[...] = jnp.maximum(o, 0.0)


def _final(u, agg, wu, wa, b):
    rb = 1000
    return pl.pallas_call(
        _fin_body,
        grid=(_U // rb,),
        in_specs=[
            pl.BlockSpec((rb, _F), lambda i: (i, 0)),
            pl.BlockSpec((rb, _D), lambda i: (i, 0)),
            pl.BlockSpec((_F, _D), lambda i: (0, 0)),
            pl.BlockSpec((_D, _D), lambda i: (0, 0)),
            pl.BlockSpec((1, _D), lambda i: (0, 0)),
        ],
        out_specs=pl.BlockSpec((rb, _D), lambda i: (i, 0)),
        out_shape=jax.ShapeDtypeStruct((_U, _D), jnp.float32),
    )(u, agg, wu, wa, b)


def kernel(u, v, e_indices, e_values, Wg1, bg1, Wg2, bg2, Wf1, bf1):
    vi = e_indices[0]
    ui = e_indices[1]
    wu = Wg1[:_F]
    wv = Wg1[_F : _F + _G]
    we = Wg1[_F + _G :]
    a32, b32 = _precompute(u, v, wu, wv, bg1.reshape(1, _HID))
    sa32, sb32 = _gather_pair(a32, b32, ui, vi)
    g = _edge_mlp(
        sa32,
        sb32,
        e_values,
        we,
        Wg2[:_HID32].astype(jnp.bfloat16),
        Wg2[_HID32:].astype(jnp.bfloat16),
        bg2.reshape(1, _D),
    )
    agg = _segment_sum(g, ui)
    return _final(
        u,
        agg,
        Wf1[:_F].astype(jnp.bfloat16),
        Wf1[_F:].astype(jnp.bfloat16),
        bf1.reshape(1, _D),
    )


# speedup vs baseline: 5.7970x; 1.2621x over previous
import functools

import jax
import jax.numpy as jnp
from jax import lax
from jax.experimental import pallas as pl
from jax.experimental.pallas import tpu as pltpu
from jax.experimental.pallas import tpu_sc as plsc

_U, _V, _E = 10000, 10000, 160000
_F, _G, _H, _D = 256, 256, 16, 256
_HID = 512

_NC, _NS, _L = 2, 16, 16
_NW = _NC * _NS

_EPW = _E // _NW
_KG = 40
_NBG = _EPW // _KG

_CH = _D // _NC
_EPS = _E // _NS
_K2 = 80
_NB2 = _EPS // _K2
_ACC = 10240
_RPS = _ACC // _NS
_TAIL = _U - (_NS - 1) * _RPS


def _sc_mesh():
    return plsc.VectorSubcoreMesh(
        core_axis_name="c", subcore_axis_name="s", num_cores=_NC, num_subcores=_NS
    )


_HID32 = _HID // 2


def _pack_bf16_pair(x):
    u = pltpu.bitcast(x, jnp.int32)
    r = u + jnp.int32(0x7FFF) + ((u >> 16) & 1)
    lo = r[:, :_HID32]
    hi = r[:, _HID32:]
    return ((lo >> 16) & jnp.int32(0xFFFF)) | (hi & jnp.int32(-65536))


def _unpack_lo(x):
    return pltpu.bitcast(x << 16, jnp.float32)


def _unpack_hi(x):
    return pltpu.bitcast(x & jnp.int32(-65536), jnp.float32)


def _pre_body(u_ref, v_ref, wu_ref, wv_ref, b1_ref, a_ref, b_ref):
    a_ref[...] = _pack_bf16_pair(
        jnp.dot(u_ref[...], wu_ref[...], preferred_element_type=jnp.float32)
        + b1_ref[...]
    )
    b_ref[...] = _pack_bf16_pair(
        jnp.dot(v_ref[...], wv_ref[...], preferred_element_type=jnp.float32)
    )


def _precompute(u, v, wu, wv, b1):
    rb = 1000
    return pl.pallas_call(
        _pre_body,
        grid=(_U // rb,),
        in_specs=[
            pl.BlockSpec((rb, _F), lambda i: (i, 0)),
            pl.BlockSpec((rb, _G), lambda i: (i, 0)),
            pl.BlockSpec((_F, _HID), lambda i: (0, 0)),
            pl.BlockSpec((_G, _HID), lambda i: (0, 0)),
            pl.BlockSpec((1, _HID), lambda i: (0, 0)),
        ],
        out_specs=[
            pl.BlockSpec((rb, _HID32), lambda i: (i, 0)),
            pl.BlockSpec((rb, _HID32), lambda i: (i, 0)),
        ],
        out_shape=[
            jax.ShapeDtypeStruct((_U, _HID32), jnp.int32),
            jax.ShapeDtypeStruct((_V, _HID32), jnp.int32),
        ],
    )(u, v, wu, wv, b1)


def _gather_body(
    a_hbm, b_hbm, ui_hbm, vi_hbm, sa_hbm, sb_hbm,
    idxu, idxv, ra, rb, sga, sgb, sst
):
    wid = lax.axis_index("s") * _NC + lax.axis_index("c")
    base = pl.multiple_of(wid * _EPW, _EPW)
    pltpu.sync_copy(ui_hbm.at[pl.ds(base, _EPW)], idxu)
    pltpu.sync_copy(vi_hbm.at[pl.ds(base, _EPW)], idxv)

    def start(j, k):
        boff = pl.multiple_of(j * _KG, _KG)
        pltpu.async_copy(a_hbm.at[idxu.at[pl.ds(boff, _KG)]], ra[k], sga[k])
        pltpu.async_copy(b_hbm.at[idxv.at[pl.ds(boff, _KG)]], rb[k], sgb[k])

    def finish(j, k):
        off = pl.multiple_of(base + j * _KG, _KG)
        boff = pl.multiple_of(j * _KG, _KG)
        pltpu.make_async_copy(a_hbm.at[idxu.at[pl.ds(boff, _KG)]], ra[k], sga[k]).wait()
        pltpu.make_async_copy(b_hbm.at[idxv.at[pl.ds(boff, _KG)]], rb[k], sgb[k]).wait()
        pltpu.async_copy(ra[k], sa_hbm.at[pl.ds(off, _KG)], sst[k])
        pltpu.async_copy(rb[k], sb_hbm.at[pl.ds(off, _KG)], sst[k])

    def drain(k):
        pltpu.make_async_copy(ra[k], sa_hbm.at[pl.ds(0, _KG)], sst[k]).wait()
        pltpu.make_async_copy(rb[k], sb_hbm.at[pl.ds(0, _KG)], sst[k]).wait()

    start(0, 0)

    def pair(p, carry):
        j0 = p * 2

        @pl.when(p > 0)
        def _():
            drain(1)

        start(j0 + 1, 1)
        finish(j0, 0)
        drain(0)
        start(j0 + 2, 0)
        finish(j0 + 1, 1)
        return carry

    lax.fori_loop(0, (_NBG - 1) // 2, pair, 0)
    finish(_NBG - 1, 0)
    drain(1)
    drain(0)


def _gather_pair(a, b, ui, vi):
    def body(a_hbm, b_hbm, ui_hbm, vi_hbm, sa_hbm, sb_hbm, iu, iv,
             ra0, ra1, rb0, rb1, sa0, sa1, sb0, sb1, ss0, ss1):
        _gather_body(
            a_hbm, b_hbm, ui_hbm, vi_hbm, sa_hbm, sb_hbm,
            iu, iv, [ra0, ra1], [rb0, rb1],
            [sa0, sa1], [sb0, sb1], [ss0, ss1],
        )

    fn = pl.kernel(
        body,
        out_type=[
            jax.ShapeDtypeStruct((_E, _HID32), jnp.int32),
            jax.ShapeDtypeStruct((_E, _HID32), jnp.int32),
        ],
        mesh=_sc_mesh(),
        scratch_types=[
            pltpu.VMEM((_EPW,), jnp.int32),
            pltpu.VMEM((_EPW,), jnp.int32),
            pltpu.VMEM((_KG, _HID32), jnp.int32),
            pltpu.VMEM((_KG, _HID32), jnp.int32),
            pltpu.VMEM((_KG, _HID32), jnp.int32),
            pltpu.VMEM((_KG, _HID32), jnp.int32),
            pltpu.SemaphoreType.DMA,
            pltpu.SemaphoreType.DMA,
            pltpu.SemaphoreType.DMA,
            pltpu.SemaphoreType.DMA,
            pltpu.SemaphoreType.DMA,
            pltpu.SemaphoreType.DMA,
        ],
    )
    return fn(a, b, ui, vi)


def _mlp_body(sa_ref, sb_ref, ev_ref, we_ref, w2lo_ref, w2hi_ref, b2_ref, g_ref):
    ew = jnp.dot(ev_ref[...], we_ref[...], preferred_element_type=jnp.float32)
    xa = sa_ref[...]
    xb = sb_ref[...]
    hlo = _unpack_lo(xa) + _unpack_lo(xb) + ew[:, :_HID32]
    hhi = _unpack_hi(xa) + _unpack_hi(xb) + ew[:, _HID32:]
    hlo = jnp.maximum(hlo, 0.0).astype(jnp.bfloat16)
    hhi = jnp.maximum(hhi, 0.0).astype(jnp.bfloat16)
    g = (
        jnp.dot(hlo, w2lo_ref[...], preferred_element_type=jnp.float32)
        + jnp.dot(hhi, w2hi_ref[...], preferred_element_type=jnp.float32)
        + b2_ref[...]
    )
    g_ref[...] = jnp.maximum(g, 0.0)


def _edge_mlp(sa, sb, ev, we, w2lo, w2hi, b2):
    be = 1280
    return pl.pallas_call(
        _mlp_body,
        grid=(_E // be,),
        in_specs=[
            pl.BlockSpec((be, _HID32), lambda i: (i, 0)),
            pl.BlockSpec((be, _HID32), lambda i: (i, 0)),
            pl.BlockSpec((be, _H), lambda i: (i, 0)),
            pl.BlockSpec((_H, _HID), lambda i: (0, 0)),
            pl.BlockSpec((_HID32, _D), lambda i: (0, 0)),
            pl.BlockSpec((_HID32, _D), lambda i: (0, 0)),
            pl.BlockSpec((1, _D), lambda i: (0, 0)),
        ],
        out_specs=pl.BlockSpec((be, _D), lambda i: (i, 0)),
        out_shape=jax.ShapeDtypeStruct((_E, _D), jnp.float32),
    )(sa, sb, ev, we, w2lo, w2hi, b2)


def _scatter_body(g_hbm, ui_hbm, out_hbm, idx, rows, acc, sld, sli):
    cid = lax.axis_index("c")
    sid = lax.axis_index("s")
    col = pl.multiple_of(cid * _CH, _CH)
    rbase = pl.multiple_of(sid * _RPS, _RPS)
    ebase = pl.multiple_of(sid * _EPS, _EPS)
    zero = jnp.zeros((_L,), jnp.float32)

    def zrow(r, carry):
        for c in range(_CH // _L):
            rows[0][r, pl.ds(c * _L, _L)] = zero
        return carry

    lax.fori_loop(0, _K2, zrow, 0)
    for k in range(_RPS // _K2):
        pltpu.sync_copy(rows[0], acc.at[pl.ds(rbase + k * _K2, _K2)])
    plsc.subcore_barrier()

    def start(j, k):
        off = pl.multiple_of(ebase + j * _K2, _K2)
        pltpu.async_copy(ui_hbm.at[pl.ds(off, _K2)], idx[k], sli[k])
        pltpu.async_copy(
            g_hbm.at[pl.ds(off, _K2), pl.ds(col, _CH)], rows[k], sld[k]
        )

    def finish(j, k):
        pltpu.make_async_copy(ui_hbm.at[pl.ds(0, _K2)], idx[k], sli[k]).wait()
        pltpu.make_async_copy(
            g_hbm.at[pl.ds(0, _K2), pl.ds(col, _CH)], rows[k], sld[k]
        ).wait()
        pltpu.sync_copy(rows[k], acc.at[idx[k]], add=True)

    start(0, 0)

    def pair(p, carry):
        j0 = p * 2
        start(j0 + 1, 1)
        finish(j0, 0)
        start(j0 + 2, 0)
        finish(j0 + 1, 1)
        return carry

    lax.fori_loop(0, (_NB2 - 1) // 2, pair, 0)
    finish(_NB2 - 1, 0)
    plsc.subcore_barrier()

    @pl.when(sid < _NS - 1)
    def _copy_full():
        pltpu.sync_copy(
            acc.at[pl.ds(rbase, _RPS)], out_hbm.at[pl.ds(rbase, _RPS), pl.ds(col, _CH)]
        )

    @pl.when(sid == _NS - 1)
    def _copy_tail():
        tb = (_NS - 1) * _RPS
        pltpu.sync_copy(
            acc.at[pl.ds(tb, _TAIL)], out_hbm.at[pl.ds(tb, _TAIL), pl.ds(col, _CH)]
        )


def _segment_sum(g, ui):
    def body(g_hbm, ui_hbm, out_hbm, i0, i1, r0, r1, acc, d0, d1, li0, li1):
        _scatter_body(
            g_hbm, ui_hbm, out_hbm, [i0, i1], [r0, r1], acc,
            [d0, d1], [li0, li1],
        )

    fn = pl.kernel(
        body,
        out_type=jax.ShapeDtypeStruct((_U, _D), jnp.float32),
        mesh=_sc_mesh(),
        scratch_types=[
            pltpu.VMEM((_K2,), jnp.int32),
            pltpu.VMEM((_K2,), jnp.int32),
            pltpu.VMEM((_K2, _CH), jnp.float32),
            pltpu.VMEM((_K2, _CH), jnp.float32),
            pltpu.VMEM_SHARED((_ACC, _CH), jnp.float32),
            pltpu.SemaphoreType.DMA,
            pltpu.SemaphoreType.DMA,
            pltpu.SemaphoreType.DMA,
            pltpu.SemaphoreType.DMA,
        ],
    )
    return fn(g, ui)


def _fin_body(u_ref, agg_ref, wu_ref, wa_ref, b_ref, o_ref):
    o = (
        jnp.dot(
            u_ref[...].astype(jnp.bfloat16),
            wu_ref[...],
            preferred_element_type=jnp.float32,
        )
        + jnp.dot(
            agg_ref[...].astype(jnp.bfloat16),
            wa_ref[...],
            preferred_element_type=jnp.float32,
        )
        + b_ref[...]
    )
    o_ref[...] = jnp.maximum(o, 0.0)


def _final(u, agg, wu, wa, b):
    rb = 1000
    return pl.pallas_call(
        _fin_body,
        grid=(_U // rb,),
        in_specs=[
            pl.BlockSpec((rb, _F), lambda i: (i, 0)),
            pl.BlockSpec((rb, _D), lambda i: (i, 0)),
            pl.BlockSpec((_F, _D), lambda i: (0, 0)),
            pl.BlockSpec((_D, _D), lambda i: (0, 0)),
            pl.BlockSpec((1, _D), lambda i: (0, 0)),
        ],
        out_specs=pl.BlockSpec((rb, _D), lambda i: (i, 0)),
        out_shape=jax.ShapeDtypeStruct((_U, _D), jnp.float32),
    )(u, agg, wu, wa, b)


def kernel(u, v, e_indices, e_values, Wg1, bg1, Wg2, bg2, Wf1, bf1):
    vi = e_indices[0]
    ui = e_indices[1]
    wu = Wg1[:_F]
    wv = Wg1[_F : _F + _G]
    we = Wg1[_F + _G :]
    a32, b32 = _precompute(u, v, wu, wv, bg1.reshape(1, _HID))
    sa32, sb32 = _gather_pair(a32, b32, ui, vi)
    g = _edge_mlp(
        sa32,
        sb32,
        e_values,
        we,
        Wg2[:_HID32].astype(jnp.bfloat16),
        Wg2[_HID32:].astype(jnp.bfloat16),
        bg2.reshape(1, _D),
    )
    agg = _segment_sum(g, ui)
    return _final(
        u,
        agg,
        Wf1[:_F].astype(jnp.bfloat16),
        Wf1[_F:].astype(jnp.bfloat16),
        bf1.reshape(1, _D),
    )


# speedup vs baseline: 5.9045x; 1.0185x over previous
import functools

import jax
import jax.numpy as jnp
from jax import lax
from jax.experimental import pallas as pl
from jax.experimental.pallas import tpu as pltpu
from jax.experimental.pallas import tpu_sc as plsc

_U, _V, _E = 10000, 10000, 160000
_F, _G, _H, _D = 256, 256, 16, 256
_HID = 512

_NC, _NS, _L = 2, 16, 16
_NW = _NC * _NS

_CM = 5
_CE = _E // _CM

_EPW = _CE // _NW
_KG = 40
_NBG = _EPW // _KG

_CH = _D // _NC
_EPS = _CE // _NS
_K2 = 80
_NB2 = _EPS // _K2
_ACC = 10240
_RPS = _ACC // _NS
_TAIL = _U - (_NS - 1) * _RPS


def _sc_mesh():
    return plsc.VectorSubcoreMesh(
        core_axis_name="c", subcore_axis_name="s", num_cores=_NC, num_subcores=_NS
    )


_HID32 = _HID // 2


def _pack_bf16_pair(x):
    u = pltpu.bitcast(x, jnp.int32)
    r = u + jnp.int32(0x7FFF) + ((u >> 16) & 1)
    lo = r[:, :_HID32]
    hi = r[:, _HID32:]
    return ((lo >> 16) & jnp.int32(0xFFFF)) | (hi & jnp.int32(-65536))


def _unpack_lo(x):
    return pltpu.bitcast(x << 16, jnp.float32)


def _unpack_hi(x):
    return pltpu.bitcast(x & jnp.int32(-65536), jnp.float32)


def _pre_body(u_ref, v_ref, wu_ref, wv_ref, b1_ref, a_ref, b_ref):
    a_ref[...] = _pack_bf16_pair(
        jnp.dot(u_ref[...], wu_ref[...], preferred_element_type=jnp.float32)
        + b1_ref[...]
    )
    b_ref[...] = _pack_bf16_pair(
        jnp.dot(v_ref[...], wv_ref[...], preferred_element_type=jnp.float32)
    )


def _precompute(u, v, wu, wv, b1):
    rb = 1000
    return pl.pallas_call(
        _pre_body,
        grid=(_U // rb,),
        in_specs=[
            pl.BlockSpec((rb, _F), lambda i: (i, 0)),
            pl.BlockSpec((rb, _G), lambda i: (i, 0)),
            pl.BlockSpec((_F, _HID), lambda i: (0, 0)),
            pl.BlockSpec((_G, _HID), lambda i: (0, 0)),
            pl.BlockSpec((1, _HID), lambda i: (0, 0)),
        ],
        out_specs=[
            pl.BlockSpec((rb, _HID32), lambda i: (i, 0)),
            pl.BlockSpec((rb, _HID32), lambda i: (i, 0)),
        ],
        out_shape=[
            jax.ShapeDtypeStruct((_U, _HID32), jnp.int32),
            jax.ShapeDtypeStruct((_V, _HID32), jnp.int32),
        ],
    )(u, v, wu, wv, b1)


def _gather_body(
    a_hbm, b_hbm, ui_hbm, vi_hbm, sa_hbm, sb_hbm,
    idxu, idxv, ra, rb, sga, sgb, sst
):
    wid = lax.axis_index("s") * _NC + lax.axis_index("c")
    base = pl.multiple_of(wid * _EPW, _EPW)
    pltpu.sync_copy(ui_hbm.at[pl.ds(base, _EPW)], idxu)
    pltpu.sync_copy(vi_hbm.at[pl.ds(base, _EPW)], idxv)

    def start(j, k):
        boff = pl.multiple_of(j * _KG, _KG)
        pltpu.async_copy(a_hbm.at[idxu.at[pl.ds(boff, _KG)]], ra[k], sga[k])
        pltpu.async_copy(b_hbm.at[idxv.at[pl.ds(boff, _KG)]], rb[k], sgb[k])

    def finish(j, k):
        off = pl.multiple_of(base + j * _KG, _KG)
        boff = pl.multiple_of(j * _KG, _KG)
        pltpu.make_async_copy(a_hbm.at[idxu.at[pl.ds(boff, _KG)]], ra[k], sga[k]).wait()
        pltpu.make_async_copy(b_hbm.at[idxv.at[pl.ds(boff, _KG)]], rb[k], sgb[k]).wait()
        pltpu.async_copy(ra[k], sa_hbm.at[pl.ds(off, _KG)], sst[k])
        pltpu.async_copy(rb[k], sb_hbm.at[pl.ds(off, _KG)], sst[k])

    def drain(k):
        pltpu.make_async_copy(ra[k], sa_hbm.at[pl.ds(0, _KG)], sst[k]).wait()
        pltpu.make_async_copy(rb[k], sb_hbm.at[pl.ds(0, _KG)], sst[k]).wait()

    start(0, 0)

    def pair(p, carry):
        j0 = p * 2

        @pl.when(p > 0)
        def _():
            drain(1)

        start(j0 + 1, 1)
        finish(j0, 0)
        drain(0)
        start(j0 + 2, 0)
        finish(j0 + 1, 1)
        return carry

    lax.fori_loop(0, (_NBG - 1) // 2, pair, 0)
    finish(_NBG - 1, 0)
    drain(1)
    drain(0)


def _gather_pair(a, b, ui, vi):
    def body(a_hbm, b_hbm, ui_hbm, vi_hbm, sa_hbm, sb_hbm, iu, iv,
             ra0, ra1, rb0, rb1, sa0, sa1, sb0, sb1, ss0, ss1):
        _gather_body(
            a_hbm, b_hbm, ui_hbm, vi_hbm, sa_hbm, sb_hbm,
            iu, iv, [ra0, ra1], [rb0, rb1],
            [sa0, sa1], [sb0, sb1], [ss0, ss1],
        )

    fn = pl.kernel(
        body,
        out_type=[
            jax.ShapeDtypeStruct((_CE, _HID32), jnp.int32),
            jax.ShapeDtypeStruct((_CE, _HID32), jnp.int32),
        ],
        mesh=_sc_mesh(),
        scratch_types=[
            pltpu.VMEM((_EPW,), jnp.int32),
            pltpu.VMEM((_EPW,), jnp.int32),
            pltpu.VMEM((_KG, _HID32), jnp.int32),
            pltpu.VMEM((_KG, _HID32), jnp.int32),
            pltpu.VMEM((_KG, _HID32), jnp.int32),
            pltpu.VMEM((_KG, _HID32), jnp.int32),
            pltpu.SemaphoreType.DMA,
            pltpu.SemaphoreType.DMA,
            pltpu.SemaphoreType.DMA,
            pltpu.SemaphoreType.DMA,
            pltpu.SemaphoreType.DMA,
            pltpu.SemaphoreType.DMA,
        ],
    )
    return fn(a, b, ui, vi)


def _mlp_body(sa_ref, sb_ref, ev_ref, we_ref, w2lo_ref, w2hi_ref, b2_ref, g_ref):
    ew = jnp.dot(ev_ref[...], we_ref[...], preferred_element_type=jnp.float32)
    xa = sa_ref[...]
    xb = sb_ref[...]
    hlo = _unpack_lo(xa) + _unpack_lo(xb) + ew[:, :_HID32]
    hhi = _unpack_hi(xa) + _unpack_hi(xb) + ew[:, _HID32:]
    hlo = jnp.maximum(hlo, 0.0).astype(jnp.bfloat16)
    hhi = jnp.maximum(hhi, 0.0).astype(jnp.bfloat16)
    g = (
        jnp.dot(hlo, w2lo_ref[...], preferred_element_type=jnp.float32)
        + jnp.dot(hhi, w2hi_ref[...], preferred_element_type=jnp.float32)
        + b2_ref[...]
    )
    g_ref[...] = jnp.maximum(g, 0.0)


def _edge_mlp(sa, sb, ev, we, w2lo, w2hi, b2):
    be = 1280
    return pl.pallas_call(
        _mlp_body,
        grid=(_CE // be,),
        in_specs=[
            pl.BlockSpec((be, _HID32), lambda i: (i, 0)),
            pl.BlockSpec((be, _HID32), lambda i: (i, 0)),
            pl.BlockSpec((be, _H), lambda i: (i, 0)),
            pl.BlockSpec((_H, _HID), lambda i: (0, 0)),
            pl.BlockSpec((_HID32, _D), lambda i: (0, 0)),
            pl.BlockSpec((_HID32, _D), lambda i: (0, 0)),
            pl.BlockSpec((1, _D), lambda i: (0, 0)),
        ],
        out_specs=pl.BlockSpec((be, _D), lambda i: (i, 0)),
        out_shape=jax.ShapeDtypeStruct((_CE, _D), jnp.float32),
    )(sa, sb, ev, we, w2lo, w2hi, b2)


def _scatter_body(g_hbm, ui_hbm, out_hbm, idx, rows, acc, sld, sli):
    cid = lax.axis_index("c")
    sid = lax.axis_index("s")
    col = pl.multiple_of(cid * _CH, _CH)
    rbase = pl.multiple_of(sid * _RPS, _RPS)
    ebase = pl.multiple_of(sid * _EPS, _EPS)
    zero = jnp.zeros((_L,), jnp.float32)

    def zrow(r, carry):
        for c in range(_CH // _L):
            rows[0][r, pl.ds(c * _L, _L)] = zero
        return carry

    lax.fori_loop(0, _K2, zrow, 0)
    for k in range(_RPS // _K2):
        pltpu.sync_copy(rows[0], acc.at[pl.ds(rbase + k * _K2, _K2)])
    plsc.subcore_barrier()

    def start(j, k):
        off = pl.multiple_of(ebase + j * _K2, _K2)
        pltpu.async_copy(ui_hbm.at[pl.ds(off, _K2)], idx[k], sli[k])
        pltpu.async_copy(
            g_hbm.at[pl.ds(off, _K2), pl.ds(col, _CH)], rows[k], sld[k]
        )

    def finish(j, k):
        pltpu.make_async_copy(ui_hbm.at[pl.ds(0, _K2)], idx[k], sli[k]).wait()
        pltpu.make_async_copy(
            g_hbm.at[pl.ds(0, _K2), pl.ds(col, _CH)], rows[k], sld[k]
        ).wait()
        pltpu.sync_copy(rows[k], acc.at[idx[k]], add=True)

    start(0, 0)

    def pair(p, carry):
        j0 = p * 2
        start(j0 + 1, 1)
        finish(j0, 0)
        start(j0 + 2, 0)
        finish(j0 + 1, 1)
        return carry

    lax.fori_loop(0, (_NB2 - 1) // 2, pair, 0)
    finish(_NB2 - 1, 0)
    plsc.subcore_barrier()

    @pl.when(sid < _NS - 1)
    def _copy_full():
        pltpu.sync_copy(
            acc.at[pl.ds(rbase, _RPS)], out_hbm.at[pl.ds(rbase, _RPS), pl.ds(col, _CH)]
        )

    @pl.when(sid == _NS - 1)
    def _copy_tail():
        tb = (_NS - 1) * _RPS
        pltpu.sync_copy(
            acc.at[pl.ds(tb, _TAIL)], out_hbm.at[pl.ds(tb, _TAIL), pl.ds(col, _CH)]
        )


def _segment_sum(g, ui):
    def body(g_hbm, ui_hbm, out_hbm, i0, i1, r0, r1, acc, d0, d1, li0, li1):
        _scatter_body(
            g_hbm, ui_hbm, out_hbm, [i0, i1], [r0, r1], acc,
            [d0, d1], [li0, li1],
        )

    fn = pl.kernel(
        body,
        out_type=jax.ShapeDtypeStruct((_U, _D), jnp.float32),
        mesh=_sc_mesh(),
        scratch_types=[
            pltpu.VMEM((_K2,), jnp.int32),
            pltpu.VMEM((_K2,), jnp.int32),
            pltpu.VMEM((_K2, _CH), jnp.float32),
            pltpu.VMEM((_K2, _CH), jnp.float32),
            pltpu.VMEM_SHARED((_ACC, _CH), jnp.float32),
            pltpu.SemaphoreType.DMA,
            pltpu.SemaphoreType.DMA,
            pltpu.SemaphoreType.DMA,
            pltpu.SemaphoreType.DMA,
        ],
    )
    return fn(g, ui)


def _fin_body(u_ref, *rest):
    agg_refs = rest[: _CM]
    wu_ref, wa_ref, b_ref, o_ref = rest[_CM:]
    agg = agg_refs[0][...]
    for r in agg_refs[1:]:
        agg = agg + r[...]
    o = (
        jnp.dot(
            u_ref[...].astype(jnp.bfloat16),
            wu_ref[...],
            preferred_element_type=jnp.float32,
        )
        + jnp.dot(
            agg.astype(jnp.bfloat16),
            wa_ref[...],
            preferred_element_type=jnp.float32,
        )
        + b_ref[...]
    )
    o_ref[...] = jnp.maximum(o, 0.0)


def _final(u, aggs, wu, wa, b):
    rb = 1000
    return pl.pallas_call(
        _fin_body,
        grid=(_U // rb,),
        in_specs=[pl.BlockSpec((rb, _F), lambda i: (i, 0))]
        + [pl.BlockSpec((rb, _D), lambda i: (i, 0)) for _ in range(_CM)]
        + [
            pl.BlockSpec((_F, _D), lambda i: (0, 0)),
            pl.BlockSpec((_D, _D), lambda i: (0, 0)),
            pl.BlockSpec((1, _D), lambda i: (0, 0)),
        ],
        out_specs=pl.BlockSpec((rb, _D), lambda i: (i, 0)),
        out_shape=jax.ShapeDtypeStruct((_U, _D), jnp.float32),
    )(u, *aggs, wu, wa, b)


def kernel(u, v, e_indices, e_values, Wg1, bg1, Wg2, bg2, Wf1, bf1):
    vi = e_indices[0]
    ui = e_indices[1]
    wu = Wg1[:_F]
    wv = Wg1[_F : _F + _G]
    we = Wg1[_F + _G :]
    a32, b32 = _precompute(u, v, wu, wv, bg1.reshape(1, _HID))
    w2lo = Wg2[:_HID32].astype(jnp.bfloat16)
    w2hi = Wg2[_HID32:].astype(jnp.bfloat16)
    b2 = bg2.reshape(1, _D)
    aggs = []
    for c in range(_CM):
        sl = slice(c * _CE, (c + 1) * _CE)
        ui_c = ui[sl]
        sa32, sb32 = _gather_pair(a32, b32, ui_c, vi[sl])
        g = _edge_mlp(sa32, sb32, e_values[sl], we, w2lo, w2hi, b2)
        aggs.append(_segment_sum(g, ui_c))
    return _final(
        u,
        aggs,
        Wf1[:_F].astype(jnp.bfloat16),
        Wf1[_F:].astype(jnp.bfloat16),
        bf1.reshape(1, _D),
    )


# speedup vs baseline: 5.9047x; 1.0000x over previous
import functools

import jax
import jax.numpy as jnp
from jax import lax
from jax.experimental import pallas as pl
from jax.experimental.pallas import tpu as pltpu
from jax.experimental.pallas import tpu_sc as plsc

_U, _V, _E = 10000, 10000, 160000
_F, _G, _H, _D = 256, 256, 16, 256
_HID = 512

_NC, _NS, _L = 2, 16, 16
_NW = _NC * _NS

_CM = 5
_CE = _E // _CM

_EPW = _CE // _NW
_KG = 40
_NBG = _EPW // _KG

_CH = _D // _NC
_EPS = _CE // _NS
_K2 = 80
_NB2 = _EPS // _K2
_ACC = 10240
_RPS = _ACC // _NS
_TAIL = _U - (_NS - 1) * _RPS


def _sc_mesh():
    return plsc.VectorSubcoreMesh(
        core_axis_name="c", subcore_axis_name="s", num_cores=_NC, num_subcores=_NS
    )


_HID32 = _HID // 2


def _pack_bf16_pair(x):
    u = pltpu.bitcast(x, jnp.int32)
    r = u + jnp.int32(0x7FFF) + ((u >> 16) & 1)
    lo = r[:, :_HID32]
    hi = r[:, _HID32:]
    return ((lo >> 16) & jnp.int32(0xFFFF)) | (hi & jnp.int32(-65536))


def _unpack_lo(x):
    return pltpu.bitcast(x << 16, jnp.float32)


def _unpack_hi(x):
    return pltpu.bitcast(x & jnp.int32(-65536), jnp.float32)


def _pre_body(u_ref, v_ref, wu_ref, wv_ref, b1_ref, a_ref, b_ref):
    a_ref[...] = _pack_bf16_pair(
        jnp.dot(u_ref[...], wu_ref[...], preferred_element_type=jnp.float32)
        + b1_ref[...]
    )
    b_ref[...] = _pack_bf16_pair(
        jnp.dot(v_ref[...], wv_ref[...], preferred_element_type=jnp.float32)
    )


def _precompute(u, v, wu, wv, b1):
    rb = 1000
    return pl.pallas_call(
        _pre_body,
        grid=(_U // rb,),
        in_specs=[
            pl.BlockSpec((rb, _F), lambda i: (i, 0)),
            pl.BlockSpec((rb, _G), lambda i: (i, 0)),
            pl.BlockSpec((_F, _HID), lambda i: (0, 0)),
            pl.BlockSpec((_G, _HID), lambda i: (0, 0)),
            pl.BlockSpec((1, _HID), lambda i: (0, 0)),
        ],
        out_specs=[
            pl.BlockSpec((rb, _HID32), lambda i: (i, 0)),
            pl.BlockSpec((rb, _HID32), lambda i: (i, 0)),
        ],
        out_shape=[
            jax.ShapeDtypeStruct((_U, _HID32), jnp.int32),
            jax.ShapeDtypeStruct((_V, _HID32), jnp.int32),
        ],
    )(u, v, wu, wv, b1)


def _gather_body(
    a_hbm, b_hbm, ui_hbm, vi_hbm, sa_hbm, sb_hbm,
    idxu, idxv, ra, rb, sga, sgb, sst
):
    wid = lax.axis_index("s") * _NC + lax.axis_index("c")
    base = pl.multiple_of(wid * _EPW, _EPW)
    pltpu.sync_copy(ui_hbm.at[pl.ds(base, _EPW)], idxu)
    pltpu.sync_copy(vi_hbm.at[pl.ds(base, _EPW)], idxv)

    def start(j, k):
        boff = pl.multiple_of(j * _KG, _KG)
        pltpu.async_copy(a_hbm.at[idxu.at[pl.ds(boff, _KG)]], ra[k], sga[k])
        pltpu.async_copy(b_hbm.at[idxv.at[pl.ds(boff, _KG)]], rb[k], sgb[k])

    def finish(j, k):
        off = pl.multiple_of(base + j * _KG, _KG)
        boff = pl.multiple_of(j * _KG, _KG)
        pltpu.make_async_copy(a_hbm.at[idxu.at[pl.ds(boff, _KG)]], ra[k], sga[k]).wait()
        pltpu.make_async_copy(b_hbm.at[idxv.at[pl.ds(boff, _KG)]], rb[k], sgb[k]).wait()
        pltpu.async_copy(ra[k], sa_hbm.at[pl.ds(off, _KG)], sst[k])
        pltpu.async_copy(rb[k], sb_hbm.at[pl.ds(off, _KG)], sst[k])

    def drain(k):
        pltpu.make_async_copy(ra[k], sa_hbm.at[pl.ds(0, _KG)], sst[k]).wait()
        pltpu.make_async_copy(rb[k], sb_hbm.at[pl.ds(0, _KG)], sst[k]).wait()

    start(0, 0)
    start(1, 1)

    def tri(t, carry):
        for d in range(3):
            j = t * 3 + d
            finish(j, d)
            jn = j + 2
            kp2 = (d + 2) % 3

            @pl.when(jnp.logical_and(jn < _NBG, j > 0))
            def _():
                drain(kp2)

            @pl.when(jn < _NBG)
            def _():
                start(jn, kp2)
        return carry

    lax.fori_loop(0, (_NBG - 1) // 3, tri, 0)
    finish(_NBG - 1, (_NBG - 1) % 3)
    drain(1)
    drain(2)
    drain(0)


def _gather_pair(a, b, ui, vi):
    def body(a_hbm, b_hbm, ui_hbm, vi_hbm, sa_hbm, sb_hbm, iu, iv,
             ra0, ra1, ra2, rb0, rb1, rb2,
             sa0, sa1, sa2, sb0, sb1, sb2, ss0, ss1, ss2):
        _gather_body(
            a_hbm, b_hbm, ui_hbm, vi_hbm, sa_hbm, sb_hbm,
            iu, iv, [ra0, ra1, ra2], [rb0, rb1, rb2],
            [sa0, sa1, sa2], [sb0, sb1, sb2], [ss0, ss1, ss2],
        )

    fn = pl.kernel(
        body,
        out_type=[
            jax.ShapeDtypeStruct((_CE, _HID32), jnp.int32),
            jax.ShapeDtypeStruct((_CE, _HID32), jnp.int32),
        ],
        mesh=_sc_mesh(),
        scratch_types=[
            pltpu.VMEM((_EPW,), jnp.int32),
            pltpu.VMEM((_EPW,), jnp.int32),
        ]
        + [pltpu.VMEM((_KG, _HID32), jnp.int32) for _ in range(6)]
        + [pltpu.SemaphoreType.DMA for _ in range(9)],
    )
    return fn(a, b, ui, vi)


def _mlp_body(sa_ref, sb_ref, ev_ref, we_ref, w2lo_ref, w2hi_ref, b2_ref, g_ref):
    ew = jnp.dot(ev_ref[...], we_ref[...], preferred_element_type=jnp.float32)
    xa = sa_ref[...]
    xb = sb_ref[...]
    hlo = _unpack_lo(xa) + _unpack_lo(xb) + ew[:, :_HID32]
    hhi = _unpack_hi(xa) + _unpack_hi(xb) + ew[:, _HID32:]
    hlo = jnp.maximum(hlo, 0.0).astype(jnp.bfloat16)
    hhi = jnp.maximum(hhi, 0.0).astype(jnp.bfloat16)
    g = (
        jnp.dot(hlo, w2lo_ref[...], preferred_element_type=jnp.float32)
        + jnp.dot(hhi, w2hi_ref[...], preferred_element_type=jnp.float32)
        + b2_ref[...]
    )
    g_ref[...] = jnp.maximum(g, 0.0)


def _edge_mlp(sa, sb, ev, we, w2lo, w2hi, b2):
    be = 1280
    return pl.pallas_call(
        _mlp_body,
        grid=(_CE // be,),
        in_specs=[
            pl.BlockSpec((be, _HID32), lambda i: (i, 0)),
            pl.BlockSpec((be, _HID32), lambda i: (i, 0)),
            pl.BlockSpec((be, _H), lambda i: (i, 0)),
            pl.BlockSpec((_H, _HID), lambda i: (0, 0)),
            pl.BlockSpec((_HID32, _D), lambda i: (0, 0)),
            pl.BlockSpec((_HID32, _D), lambda i: (0, 0)),
            pl.BlockSpec((1, _D), lambda i: (0, 0)),
        ],
        out_specs=pl.BlockSpec((be, _D), lambda i: (i, 0)),
        out_shape=jax.ShapeDtypeStruct((_CE, _D), jnp.float32),
    )(sa, sb, ev, we, w2lo, w2hi, b2)


def _scatter_body(g_hbm, ui_hbm, out_hbm, idx, rows, acc, sld, sli):
    cid = lax.axis_index("c")
    sid = lax.axis_index("s")
    col = pl.multiple_of(cid * _CH, _CH)
    rbase = pl.multiple_of(sid * _RPS, _RPS)
    ebase = pl.multiple_of(sid * _EPS, _EPS)
    zero = jnp.zeros((_L,), jnp.float32)

    def zrow(r, carry):
        for c in range(_CH // _L):
            rows[0][r, pl.ds(c * _L, _L)] = zero
        return carry

    lax.fori_loop(0, _K2, zrow, 0)
    for k in range(_RPS // _K2):
        pltpu.sync_copy(rows[0], acc.at[pl.ds(rbase + k * _K2, _K2)])
    plsc.subcore_barrier()

    def start(j, k):
        off = pl.multiple_of(ebase + j * _K2, _K2)
        pltpu.async_copy(ui_hbm.at[pl.ds(off, _K2)], idx[k], sli[k])
        pltpu.async_copy(
            g_hbm.at[pl.ds(off, _K2), pl.ds(col, _CH)], rows[k], sld[k]
        )

    def finish(j, k):
        pltpu.make_async_copy(ui_hbm.at[pl.ds(0, _K2)], idx[k], sli[k]).wait()
        pltpu.make_async_copy(
            g_hbm.at[pl.ds(0, _K2), pl.ds(col, _CH)], rows[k], sld[k]
        ).wait()
        pltpu.sync_copy(rows[k], acc.at[idx[k]], add=True)

    start(0, 0)

    def pair(p, carry):
        j0 = p * 2
        start(j0 + 1, 1)
        finish(j0, 0)
        start(j0 + 2, 0)
        finish(j0 + 1, 1)
        return carry

    lax.fori_loop(0, (_NB2 - 1) // 2, pair, 0)
    finish(_NB2 - 1, 0)
    plsc.subcore_barrier()

    @pl.when(sid < _NS - 1)
    def _copy_full():
        pltpu.sync_copy(
            acc.at[pl.ds(rbase, _RPS)], out_hbm.at[pl.ds(rbase, _RPS), pl.ds(col, _CH)]
        )

    @pl.when(sid == _NS - 1)
    def _copy_tail():
        tb = (_NS - 1) * _RPS
        pltpu.sync_copy(
            acc.at[pl.ds(tb, _TAIL)], out_hbm.at[pl.ds(tb, _TAIL), pl.ds(col, _CH)]
        )


def _segment_sum(g, ui):
    def body(g_hbm, ui_hbm, out_hbm, i0, i1, r0, r1, acc, d0, d1, li0, li1):
        _scatter_body(
            g_hbm, ui_hbm, out_hbm, [i0, i1], [r0, r1], acc,
            [d0, d1], [li0, li1],
        )

    fn = pl.kernel(
        body,
        out_type=jax.ShapeDtypeStruct((_U, _D), jnp.float32),
        mesh=_sc_mesh(),
        scratch_types=[
            pltpu.VMEM((_K2,), jnp.int32),
            pltpu.VMEM((_K2,), jnp.int32),
            pltpu.VMEM((_K2, _CH), jnp.float32),
            pltpu.VMEM((_K2, _CH), jnp.float32),
            pltpu.VMEM_SHARED((_ACC, _CH), jnp.float32),
            pltpu.SemaphoreType.DMA,
            pltpu.SemaphoreType.DMA,
            pltpu.SemaphoreType.DMA,
            pltpu.SemaphoreType.DMA,
        ],
    )
    return fn(g, ui)


def _fin_body(u_ref, *rest):
    agg_refs = rest[: _CM]
    wu_ref, wa_ref, b_ref, o_ref = rest[_CM:]
    agg = agg_refs[0][...]
    for r in agg_refs[1:]:
        agg = agg + r[...]
    o = (
        jnp.dot(
            u_ref[...].astype(jnp.bfloat16),
            wu_ref[...],
            preferred_element_type=jnp.float32,
        )
        + jnp.dot(
            agg.astype(jnp.bfloat16),
            wa_ref[...],
            preferred_element_type=jnp.float32,
        )
        + b_ref[...]
    )
    o_ref[...] = jnp.maximum(o, 0.0)


def _final(u, aggs, wu, wa, b):
    rb = 1000
    return pl.pallas_call(
        _fin_body,
        grid=(_U // rb,),
        in_specs=[pl.BlockSpec((rb, _F), lambda i: (i, 0))]
        + [pl.BlockSpec((rb, _D), lambda i: (i, 0)) for _ in range(_CM)]
        + [
            pl.BlockSpec((_F, _D), lambda i: (0, 0)),
            pl.BlockSpec((_D, _D), lambda i: (0, 0)),
            pl.BlockSpec((1, _D), lambda i: (0, 0)),
        ],
        out_specs=pl.BlockSpec((rb, _D), lambda i: (i, 0)),
        out_shape=jax.ShapeDtypeStruct((_U, _D), jnp.float32),
    )(u, *aggs, wu, wa, b)


def kernel(u, v, e_indices, e_values, Wg1, bg1, Wg2, bg2, Wf1, bf1):
    vi = e_indices[0]
    ui = e_indices[1]
    wu = Wg1[:_F]
    wv = Wg1[_F : _F + _G]
    we = Wg1[_F + _G :]
    a32, b32 = _precompute(u, v, wu, wv, bg1.reshape(1, _HID))
    w2lo = Wg2[:_HID32].astype(jnp.bfloat16)
    w2hi = Wg2[_HID32:].astype(jnp.bfloat16)
    b2 = bg2.reshape(1, _D)
    aggs = []
    for c in range(_CM):
        sl = slice(c * _CE, (c + 1) * _CE)
        ui_c = ui[sl]
        sa32, sb32 = _gather_pair(a32, b32, ui_c, vi[sl])
        g = _edge_mlp(sa32, sb32, e_values[sl], we, w2lo, w2hi, b2)
        aggs.append(_segment_sum(g, ui_c))
    return _final(
        u,
        aggs,
        Wf1[:_F].astype(jnp.bfloat16),
        Wf1[_F:].astype(jnp.bfloat16),
        bf1.reshape(1, _D),
    )
